# Initial kernel scaffold; baseline (speedup 1.0000x reference)
import jax
import jax.numpy as jnp
from jax.experimental import pallas as pl


def kernel(ui_pair, feature_index, preference_index, ui_emb_w, feature_emb_w, bias):
    raise NotImplementedError("write your pallas kernel here")


# speedup vs baseline: 1.3074x; 1.3074x over previous
import functools

import jax
import jax.numpy as jnp
from jax import lax
from jax.experimental import pallas as pl
from jax.experimental.pallas import tpu as pltpu
from jax.experimental.pallas import tpu_sc as plsc

B = 4096
HIST = 50
EMB = 64
ROW = EMB + 1
ROWP = 72
SLOTS = 2 + HIST
NC = 2
NS = 16
NW = NC * NS
BPW = B // NW
CHUNK = 32
NCHUNK = BPW // CHUNK


def _build_sc_gather():
    mesh = plsc.VectorSubcoreMesh(core_axis_name="c", subcore_axis_name="s",
                                  num_cores=NC, num_subcores=NS)

    @functools.partial(
        pl.kernel,
        mesh=mesh,
        compiler_params=pltpu.CompilerParams(needs_layout_passes=False,
                                             use_tc_tiling_on_sc=False),
        out_type=(
            jax.ShapeDtypeStruct((B, SLOTS, EMB), jnp.float32),
            jax.ShapeDtypeStruct((B, SLOTS, 1), jnp.float32),
        ),
        scratch_types=[
            pltpu.VMEM((CHUNK, SLOTS, ROWP), jnp.float32),
            pltpu.VMEM((CHUNK, 2), jnp.int32),
            pltpu.VMEM((CHUNK, HIST), jnp.int32),
            pltpu.SemaphoreType.DMA,
        ],
    )
    def sc_gather(ui_pair, pref_idx, ui_w, feat_w,
                  nz_out, biascol_out,
                  all_buf, ui_idx_v, pref_idx_v, sem):
        wid = lax.axis_index("s") * NC + lax.axis_index("c")

        for ci in range(NCHUNK):
            b0 = wid * BPW + ci * CHUNK

            pltpu.sync_copy(ui_pair.at[pl.ds(b0, CHUNK), :], ui_idx_v)
            pltpu.sync_copy(pref_idx.at[pl.ds(b0, CHUNK), :], pref_idx_v)

            def fire(i, carry):
                pltpu.async_copy(ui_w.at[ui_idx_v.at[i]],
                                 all_buf.at[i, pl.ds(0, 2), :], sem)
                pltpu.async_copy(feat_w.at[pref_idx_v.at[i]],
                                 all_buf.at[i, pl.ds(2, HIST), :], sem)
                return carry

            lax.fori_loop(0, CHUNK, fire, 0)

            def drain(i, carry):
                pltpu.make_async_copy(
                    ui_w.at[ui_idx_v.at[i]],
                    all_buf.at[i, pl.ds(0, 2), :], sem).wait()
                pltpu.make_async_copy(
                    feat_w.at[pref_idx_v.at[i]],
                    all_buf.at[i, pl.ds(2, HIST), :], sem).wait()
                return carry

            lax.fori_loop(0, CHUNK, drain, 0)

            pltpu.sync_copy(all_buf.at[:, :, pl.ds(0, EMB)],
                            nz_out.at[pl.ds(b0, CHUNK), :, :])
            pltpu.sync_copy(all_buf.at[:, :, pl.ds(EMB, 1)],
                            biascol_out.at[pl.ds(b0, CHUNK), :, :])

    return sc_gather


_SC_GATHER = _build_sc_gather()

TC_BLOCK = 512


def _tc_reduce_body(nz_ref, bias_ref, out_ref):
    nz = nz_ref[...]
    u1 = nz[:, 0, :]
    u2 = nz[:, 1, :]
    s2 = jnp.sum(nz[:, 2:, :], axis=1)
    fm = u1 * u2 + (u1 + u2) * s2
    out_ref[...] = (jnp.sum(fm, axis=1, keepdims=True)
                    + bias_ref[0])


def _tc_reduce(nz, bias):
    return pl.pallas_call(
        _tc_reduce_body,
        grid=(B // TC_BLOCK,),
        in_specs=[
            pl.BlockSpec((TC_BLOCK, SLOTS, EMB), lambda i: (i, 0, 0)),
            pl.BlockSpec(memory_space=pltpu.SMEM),
        ],
        out_specs=pl.BlockSpec((TC_BLOCK, 1), lambda i: (i, 0)),
        out_shape=jax.ShapeDtypeStruct((B, 1), jnp.float32),
    )(nz, bias)


@jax.jit
def _fm(ui_pair, preference_index, ui_emb_w, feature_emb_w, bias):
    ui_p = jnp.pad(ui_emb_w, ((0, 0), (0, ROWP - ROW)))
    feat_p = jnp.pad(feature_emb_w, ((0, 0), (0, ROWP - ROW)))
    nz, biascol = _SC_GATHER(
        ui_pair.astype(jnp.int32), preference_index.astype(jnp.int32),
        ui_p, feat_p)
    result = _tc_reduce(nz, bias)
    return result, biascol, nz


def kernel(ui_pair, feature_index, preference_index, ui_emb_w, feature_emb_w,
           bias):
    del feature_index
    return _fm(ui_pair, preference_index, ui_emb_w, feature_emb_w, bias)
